# initial kernel scaffold (unmeasured)
import jax
import jax.numpy as jnp
from jax import lax
from jax.experimental import pallas as pl
from jax.experimental.pallas import tpu as pltpu

N_DEV = 4
E_LOCAL = 8
T = 2048
D = 1024


def kernel(x, router_W, route_idx, expert_W):
    scores = jnp.dot(x, router_W, precision=lax.Precision.HIGHEST)
    m = jnp.max(scores, axis=1, keepdims=True)
    e = jnp.exp(scores - m)
    probs = e / jnp.sum(e, axis=1, keepdims=True)
    p = jnp.take_along_axis(probs, route_idx, axis=1)
    gates = p / jnp.sum(p, axis=1, keepdims=True)
    packed = jnp.concatenate(
        [gates.astype(jnp.float32), route_idx.astype(jnp.float32)], axis=1
    )

    x_bf = x.astype(jnp.bfloat16)
    ew_bf = expert_W.astype(jnp.bfloat16)

    def body(x_ref, pk_ref, ew_ref, out_ref,
             x_all, pk_all, w_buf, pacc, precv,
             ag_send, ag_recv, p_send, p_recv, load_sem):
        my = lax.axis_index("i")
        left = lax.rem(my + N_DEV - 1, N_DEV)
        right = lax.rem(my + 1, N_DEV)

        barrier = pltpu.get_barrier_semaphore()
        for nbr in (left, right):
            pl.semaphore_signal(barrier, inc=1, device_id=(nbr,),
                                device_id_type=pl.DeviceIdType.MESH)
        pl.semaphore_wait(barrier, 2)

        out_ref[...] = jnp.zeros((T, D), jnp.float32)
        for s in range(N_DEV - 1):
            pacc[s] = jnp.zeros((T, D), jnp.bfloat16)

        for h in range(N_DEV - 1):
            pairs = [
                (x_ref if h == 0 else x_all.at[h - 1], x_all.at[h], 0),
                (pk_ref if h == 0 else pk_all.at[h - 1], pk_all.at[h], 1),
            ]
            rdmas = []
            for src, dst, k in pairs:
                r = pltpu.make_async_remote_copy(
                    src_ref=src, dst_ref=dst,
                    send_sem=ag_send.at[h, k], recv_sem=ag_recv.at[h, k],
                    device_id=(right,), device_id_type=pl.DeviceIdType.MESH,
                )
                r.start()
                rdmas.append(r)
            for r in rdmas:
                r.wait()

        def chunk_update(xs, pk, e_id, wj):
            g0 = pk[:, 0:1]
            g1 = pk[:, 1:2]
            e_f = e_id.astype(jnp.float32)
            w = (jnp.where(pk[:, 2:3] == e_f, g0, 0.0)
                 + jnp.where(pk[:, 3:4] == e_f, g1, 0.0))
            y = jnp.dot(xs, wj, preferred_element_type=jnp.float32)
            return w * y

        for j in range(E_LOCAL):
            cp = pltpu.make_async_copy(ew_ref.at[j], w_buf, load_sem)
            cp.start()
            cp.wait()
            e_id = my * E_LOCAL + j
            wj = w_buf[...]
            out_ref[...] += chunk_update(x_ref[...], pk_ref[...], e_id, wj)
            for s in range(N_DEV - 1):
                upd = chunk_update(x_all[s], pk_all[s], e_id, wj)
                pacc[s] = pacc[s] + upd.astype(jnp.bfloat16)

        prdmas = []
        for s in range(N_DEV - 1):
            owner = lax.rem(my + N_DEV - 1 - s, N_DEV)
            r = pltpu.make_async_remote_copy(
                src_ref=pacc.at[s], dst_ref=precv.at[s],
                send_sem=p_send.at[s], recv_sem=p_recv.at[s],
                device_id=(owner,), device_id_type=pl.DeviceIdType.MESH,
            )
            r.start()
            prdmas.append(r)
        for r in prdmas:
            r.wait()

        out_ref[...] = out_ref[...] + (
            precv[0].astype(jnp.float32)
            + precv[1].astype(jnp.float32)
            + precv[2].astype(jnp.float32)
        )

    return pl.pallas_call(
        body,
        out_shape=jax.ShapeDtypeStruct((T, D), jnp.float32),
        in_specs=[
            pl.BlockSpec(memory_space=pltpu.VMEM),
            pl.BlockSpec(memory_space=pltpu.VMEM),
            pl.BlockSpec(memory_space=pltpu.ANY),
        ],
        out_specs=pl.BlockSpec(memory_space=pltpu.VMEM),
        scratch_shapes=[
            pltpu.VMEM((N_DEV - 1, T, D), jnp.bfloat16),
            pltpu.VMEM((N_DEV - 1, T, 4), jnp.float32),
            pltpu.VMEM((D, D), jnp.bfloat16),
            pltpu.VMEM((N_DEV - 1, T, D), jnp.bfloat16),
            pltpu.VMEM((N_DEV - 1, T, D), jnp.bfloat16),
            pltpu.SemaphoreType.DMA((N_DEV - 1, 2)),
            pltpu.SemaphoreType.DMA((N_DEV - 1, 2)),
            pltpu.SemaphoreType.DMA((N_DEV - 1,)),
            pltpu.SemaphoreType.DMA((N_DEV - 1,)),
            pltpu.SemaphoreType.DMA,
        ],
        compiler_params=pltpu.CompilerParams(
            collective_id=0,
            vmem_limit_bytes=128 * 1024 * 1024,
        ),
    )(x_bf, packed, ew_bf)


# baseline (device time: 513782 ns/iter reference)
import jax

jax.config.update("jax_compilation_cache_dir", "/tmp/scband_jax_cache")
jax.config.update("jax_persistent_cache_min_compile_time_secs", 0.0)
jax.config.update("jax_persistent_cache_min_entry_size_bytes", 0)

import jax.numpy as jnp
from jax import lax
from jax.experimental import pallas as pl
from jax.experimental.pallas import tpu as pltpu

N_DEV = 4
E_LOCAL = 8
T = 2048
D = 1024


def kernel(x, router_W, route_idx, expert_W):
    scores = jnp.dot(x, router_W, precision=lax.Precision.HIGHEST)
    m = jnp.max(scores, axis=1, keepdims=True)
    e = jnp.exp(scores - m)
    probs = e / jnp.sum(e, axis=1, keepdims=True)
    p = jnp.take_along_axis(probs, route_idx, axis=1)
    gates = p / jnp.sum(p, axis=1, keepdims=True)
    packed = jnp.concatenate(
        [gates.astype(jnp.float32), route_idx.astype(jnp.float32)], axis=1
    )

    x_bf = x.astype(jnp.bfloat16)
    ew_bf = expert_W.astype(jnp.bfloat16)

    def body(x_ref, pk_ref, ew_ref, out_ref,
             x_all, pk_all, w_buf, pacc, precv,
             ag_send, ag_recv, p_send, p_recv, load_sem):
        my = lax.axis_index("i")
        left = lax.rem(my + N_DEV - 1, N_DEV)
        right = lax.rem(my + 1, N_DEV)

        barrier = pltpu.get_barrier_semaphore()
        for nbr in (left, right):
            pl.semaphore_signal(barrier, inc=1, device_id=(nbr,),
                                device_id_type=pl.DeviceIdType.MESH)
        pl.semaphore_wait(barrier, 2)

        out_ref[...] = jnp.zeros((T, D), jnp.float32)
        for s in range(N_DEV - 1):
            pacc[s] = jnp.zeros((T, D), jnp.bfloat16)

        for h in range(N_DEV - 1):
            pairs = [
                (x_ref if h == 0 else x_all.at[h - 1], x_all.at[h], 0),
                (pk_ref if h == 0 else pk_all.at[h - 1], pk_all.at[h], 1),
            ]
            rdmas = []
            for src, dst, k in pairs:
                r = pltpu.make_async_remote_copy(
                    src_ref=src, dst_ref=dst,
                    send_sem=ag_send.at[h, k], recv_sem=ag_recv.at[h, k],
                    device_id=(right,), device_id_type=pl.DeviceIdType.MESH,
                )
                r.start()
                rdmas.append(r)
            for r in rdmas:
                r.wait()

        H = D // 2

        def tok_weight(pk, e_id):
            e_f = e_id.astype(jnp.float32)
            return (jnp.where(pk[:, 2:3] == e_f, pk[:, 0:1], 0.0)
                    + jnp.where(pk[:, 3:4] == e_f, pk[:, 1:2], 0.0))

        def expert_step(j, _):
            e_id = my * E_LOCAL + j
            w_own = tok_weight(pk_ref[...], e_id)
            w_rem = [tok_weight(pk_all[s], e_id) for s in range(N_DEV - 1)]
            for c in range(2):
                cp = pltpu.make_async_copy(
                    ew_ref.at[j, :, pl.ds(c * H, H)], w_buf, load_sem)
                cp.start()
                cp.wait()
                wj = w_buf[...]
                cols = pl.ds(c * H, H)
                y = jnp.dot(x_ref[...], wj,
                            preferred_element_type=jnp.float32)
                out_ref[:, cols] += w_own * y
                for s in range(N_DEV - 1):
                    y = jnp.dot(x_all[s], wj,
                                preferred_element_type=jnp.float32)
                    pacc[s, :, cols] = (
                        pacc[s, :, cols]
                        + (w_rem[s] * y).astype(jnp.bfloat16))
            return _

        lax.fori_loop(0, E_LOCAL, expert_step, 0)

        prdmas = []
        for s in range(N_DEV - 1):
            owner = lax.rem(my + N_DEV - 1 - s, N_DEV)
            r = pltpu.make_async_remote_copy(
                src_ref=pacc.at[s], dst_ref=precv.at[s],
                send_sem=p_send.at[s], recv_sem=p_recv.at[s],
                device_id=(owner,), device_id_type=pl.DeviceIdType.MESH,
            )
            r.start()
            prdmas.append(r)
        for r in prdmas:
            r.wait()

        out_ref[...] = out_ref[...] + (
            precv[0].astype(jnp.float32)
            + precv[1].astype(jnp.float32)
            + precv[2].astype(jnp.float32)
        )

    return pl.pallas_call(
        body,
        out_shape=jax.ShapeDtypeStruct((T, D), jnp.float32),
        in_specs=[
            pl.BlockSpec(memory_space=pltpu.VMEM),
            pl.BlockSpec(memory_space=pltpu.VMEM),
            pl.BlockSpec(memory_space=pltpu.MemorySpace.HBM),
        ],
        out_specs=pl.BlockSpec(memory_space=pltpu.VMEM),
        scratch_shapes=[
            pltpu.VMEM((N_DEV - 1, T, D), jnp.bfloat16),
            pltpu.VMEM((N_DEV - 1, T, 4), jnp.float32),
            pltpu.VMEM((D, D // 2), jnp.bfloat16),
            pltpu.VMEM((N_DEV - 1, T, D), jnp.bfloat16),
            pltpu.VMEM((N_DEV - 1, T, D), jnp.bfloat16),
            pltpu.SemaphoreType.DMA((N_DEV - 1, 2)),
            pltpu.SemaphoreType.DMA((N_DEV - 1, 2)),
            pltpu.SemaphoreType.DMA((N_DEV - 1,)),
            pltpu.SemaphoreType.DMA((N_DEV - 1,)),
            pltpu.SemaphoreType.DMA,
        ],
        compiler_params=pltpu.CompilerParams(
            collective_id=0,
            vmem_limit_bytes=128 * 1024 * 1024,
        ),
    )(x_bf, packed, ew_bf)


# device time: 318620 ns/iter; 1.6125x vs baseline; 1.6125x over previous
import jax

jax.config.update("jax_compilation_cache_dir", "/tmp/scband_jax_cache")
jax.config.update("jax_persistent_cache_min_compile_time_secs", 0.0)
jax.config.update("jax_persistent_cache_min_entry_size_bytes", 0)

import jax.numpy as jnp
from jax import lax
from jax.experimental import pallas as pl
from jax.experimental.pallas import tpu as pltpu

N_DEV = 4
E_LOCAL = 8
T = 2048
D = 1024
H = D // 2


def kernel(x, router_W, route_idx, expert_W):
    scores = jnp.dot(x, router_W, precision=lax.Precision.HIGHEST)
    m = jnp.max(scores, axis=1, keepdims=True)
    e = jnp.exp(scores - m)
    probs = e / jnp.sum(e, axis=1, keepdims=True)
    eids = jnp.arange(32, dtype=jnp.int32)[None, :]
    p0 = jnp.sum(jnp.where(route_idx[:, 0:1] == eids, probs, 0.0),
                 axis=1, keepdims=True)
    p1 = jnp.sum(jnp.where(route_idx[:, 1:2] == eids, probs, 0.0),
                 axis=1, keepdims=True)
    ps = p0 + p1
    packed = jnp.concatenate(
        [p0 / ps, p1 / ps, route_idx.astype(jnp.float32)], axis=1
    )

    x_bf = x.astype(jnp.bfloat16)
    ew_bf = expert_W.astype(jnp.bfloat16)

    def body(x_ref, pk_ref, ew_ref, out_ref,
             x_all, pk_all, w_bufs, pacc, precv,
             ag_send, ag_recv, p_send, p_recv, load_sems):
        my = lax.axis_index("i")
        left = lax.rem(my + N_DEV - 1, N_DEV)
        right = lax.rem(my + 1, N_DEV)

        barrier = pltpu.get_barrier_semaphore()
        for nbr in (left, right):
            pl.semaphore_signal(barrier, inc=1, device_id=(nbr,),
                                device_id_type=pl.DeviceIdType.MESH)
        pl.semaphore_wait(barrier, 2)

        for s in range(N_DEV - 1):
            pacc[s] = jnp.zeros((T, D), jnp.bfloat16)

        def start_hop(h):
            rs = []
            for src, dst, k in (
                (x_ref if h == 0 else x_all.at[h - 1], x_all.at[h], 0),
                (pk_ref if h == 0 else pk_all.at[h - 1], pk_all.at[h], 1),
            ):
                r = pltpu.make_async_remote_copy(
                    src_ref=src, dst_ref=dst,
                    send_sem=ag_send.at[h, k], recv_sem=ag_recv.at[h, k],
                    device_id=(right,), device_id_type=pl.DeviceIdType.MESH,
                )
                r.start()
                rs.append(r)
            return rs

        def w_copy(j, c):
            return pltpu.make_async_copy(
                ew_ref.at[j, :, pl.ds(c * H, H)],
                w_bufs.at[lax.rem(j, 2), c],
                load_sems.at[lax.rem(j, 2), c],
            )

        def compute_stage(s):
            xs_ref = x_ref if s < 0 else x_all.at[s]
            pk = pk_ref[...] if s < 0 else pk_all[s]
            for c in range(2):
                w_copy(0, c).start()

            def expert_step(j, _):
                @pl.when(j + 1 < E_LOCAL)
                def _prefetch():
                    for c in range(2):
                        w_copy(j + 1, c).start()
                e_f = (my * E_LOCAL + j).astype(jnp.float32)
                w = (jnp.where(pk[:, 2:3] == e_f, pk[:, 0:1], 0.0)
                     + jnp.where(pk[:, 3:4] == e_f, pk[:, 1:2], 0.0))
                for c in range(2):
                    w_copy(j, c).wait()
                    wj = w_bufs[lax.rem(j, 2), c]
                    cols = pl.ds(c * H, H)
                    y = jnp.dot(xs_ref[...], wj,
                                preferred_element_type=jnp.float32)
                    if s < 0:
                        out_ref[:, cols] += w * y
                    else:
                        pacc[s, :, cols] = (
                            pacc[s, :, cols] + (w * y).astype(jnp.bfloat16))
                return _

            lax.fori_loop(0, E_LOCAL, expert_step, 0)

        def start_partial(s):
            owner = lax.rem(my + N_DEV - 1 - s, N_DEV)
            r = pltpu.make_async_remote_copy(
                src_ref=pacc.at[s], dst_ref=precv.at[s],
                send_sem=p_send.at[s], recv_sem=p_recv.at[s],
                device_id=(owner,), device_id_type=pl.DeviceIdType.MESH,
            )
            r.start()
            return r

        out_ref[...] = jnp.zeros((T, D), jnp.float32)
        hop = start_hop(0)
        compute_stage(-1)
        partials = []
        for s in range(N_DEV - 1):
            for r in hop:
                r.wait()
            if s < N_DEV - 2:
                hop = start_hop(s + 1)
            compute_stage(s)
            partials.append(start_partial(s))
        for r in partials:
            r.wait()

        out_ref[...] = out_ref[...] + (
            precv[0].astype(jnp.float32)
            + precv[1].astype(jnp.float32)
            + precv[2].astype(jnp.float32)
        )

    return pl.pallas_call(
        body,
        out_shape=jax.ShapeDtypeStruct((T, D), jnp.float32),
        in_specs=[
            pl.BlockSpec(memory_space=pltpu.VMEM),
            pl.BlockSpec(memory_space=pltpu.VMEM),
            pl.BlockSpec(memory_space=pltpu.MemorySpace.HBM),
        ],
        out_specs=pl.BlockSpec(memory_space=pltpu.VMEM),
        scratch_shapes=[
            pltpu.VMEM((N_DEV - 1, T, D), jnp.bfloat16),
            pltpu.VMEM((N_DEV - 1, T, 4), jnp.float32),
            pltpu.VMEM((2, 2, D, H), jnp.bfloat16),
            pltpu.VMEM((N_DEV - 1, T, D), jnp.bfloat16),
            pltpu.VMEM((N_DEV - 1, T, D), jnp.bfloat16),
            pltpu.SemaphoreType.DMA((N_DEV - 1, 2)),
            pltpu.SemaphoreType.DMA((N_DEV - 1, 2)),
            pltpu.SemaphoreType.DMA((N_DEV - 1,)),
            pltpu.SemaphoreType.DMA((N_DEV - 1,)),
            pltpu.SemaphoreType.DMA((2, 2)),
        ],
        compiler_params=pltpu.CompilerParams(
            collective_id=0,
            vmem_limit_bytes=128 * 1024 * 1024,
        ),
    )(x_bf, packed, ew_bf)


# device time: 313566 ns/iter; 1.6385x vs baseline; 1.0161x over previous
import jax

jax.config.update("jax_compilation_cache_dir", "/tmp/scband_jax_cache")
jax.config.update("jax_persistent_cache_min_compile_time_secs", 0.0)
jax.config.update("jax_persistent_cache_min_entry_size_bytes", 0)

import jax.numpy as jnp
from jax import lax
from jax.experimental import pallas as pl
from jax.experimental.pallas import tpu as pltpu

N_DEV = 4
E_LOCAL = 8
T = 2048
D = 1024
E_HALF = 4
KH = E_HALF * D
TT = 128
N_TILES = T // TT


def kernel(x, router_W, route_idx, expert_W):
    scores = jnp.dot(x, router_W, precision=lax.Precision.HIGHEST)
    m = jnp.max(scores, axis=1, keepdims=True)
    e = jnp.exp(scores - m)
    probs = e / jnp.sum(e, axis=1, keepdims=True)
    eids = jnp.arange(32, dtype=jnp.int32)[None, :]
    p0 = jnp.sum(jnp.where(route_idx[:, 0:1] == eids, probs, 0.0),
                 axis=1, keepdims=True)
    p1 = jnp.sum(jnp.where(route_idx[:, 1:2] == eids, probs, 0.0),
                 axis=1, keepdims=True)
    ps = p0 + p1
    packed = jnp.concatenate(
        [p0 / ps, p1 / ps, route_idx.astype(jnp.float32)], axis=1
    ).astype(jnp.bfloat16)

    x_bf = x.astype(jnp.bfloat16)
    ew_bf = expert_W.astype(jnp.bfloat16).reshape(E_LOCAL * D, D)

    def body(x_ref, pk_ref, ew_ref, out_ref,
             x_all, pk_all, w_half, xw, acc_bf, pacc, precv,
             ag_send, ag_recv, p_send, p_recv, load_sem):
        my = lax.axis_index("i")
        left = lax.rem(my + N_DEV - 1, N_DEV)
        right = lax.rem(my + 1, N_DEV)

        def w_load(h):
            return pltpu.make_async_copy(
                ew_ref.at[pl.ds(h * KH, KH), :], w_half, load_sem)

        w_load(0).start()

        barrier = pltpu.get_barrier_semaphore()
        for nbr in (left, right):
            pl.semaphore_signal(barrier, inc=1, device_id=(nbr,),
                                device_id_type=pl.DeviceIdType.MESH)
        pl.semaphore_wait(barrier, 2)

        def start_hop(h):
            rs = []
            for src, dst, k in (
                (x_ref if h == 0 else x_all.at[h - 1], x_all.at[h], 0),
                (pk_ref if h == 0 else pk_all.at[h - 1], pk_all.at[h], 1),
            ):
                r = pltpu.make_async_remote_copy(
                    src_ref=src, dst_ref=dst,
                    send_sem=ag_send.at[h, k], recv_sem=ag_recv.at[h, k],
                    device_id=(right,), device_id_type=pl.DeviceIdType.MESH,
                )
                r.start()
                rs.append(r)
            return rs

        def compute_stage(s, halves, wait_first=False):
            xs_ref = x_ref if s < 0 else x_all.at[s]
            pk_s = pk_ref if s < 0 else pk_all.at[s]
            for hi, h in enumerate(halves):
                if hi > 0:
                    w_load(h).start()
                    w_load(h).wait()
                elif wait_first:
                    w_load(h).wait()

                def tile_step(t, _, hi=hi, h=h):
                    rows = pl.ds(t * TT, TT)
                    b = lax.rem(t, 2)
                    xs_t = xs_ref[rows, :]
                    pk_t = pk_s[rows, :]
                    for jj in range(E_HALF):
                        e_f = (my * E_LOCAL + h * E_HALF + jj).astype(
                            jnp.bfloat16)
                        w = (jnp.where(pk_t[:, 2:3] == e_f, pk_t[:, 0:1], 0)
                             + jnp.where(pk_t[:, 3:4] == e_f, pk_t[:, 1:2],
                                         0))
                        xw[b, :, jj * D:(jj + 1) * D] = xs_t * w
                    y = jnp.dot(xw[b], w_half[...],
                                preferred_element_type=jnp.float32)
                    if hi == 0:
                        acc_bf[rows, :] = y.astype(jnp.bfloat16)
                    else:
                        tot = (acc_bf[rows, :].astype(jnp.float32)
                               + y).astype(jnp.bfloat16)
                        if s < 0:
                            out_ref[rows, :] = tot
                        else:
                            pacc[s, rows, :] = tot
                    return _

                lax.fori_loop(0, N_TILES, tile_step, 0)

        def start_partial(s):
            owner = lax.rem(my + N_DEV - 1 - s, N_DEV)
            r = pltpu.make_async_remote_copy(
                src_ref=pacc.at[s], dst_ref=precv.at[s],
                send_sem=p_send.at[s], recv_sem=p_recv.at[s],
                device_id=(owner,), device_id_type=pl.DeviceIdType.MESH,
            )
            r.start()
            return r

        hop = start_hop(0)
        compute_stage(-1, (0, 1), wait_first=True)
        partials = []
        for s in range(N_DEV - 1):
            for r in hop:
                r.wait()
            if s < N_DEV - 2:
                hop = start_hop(s + 1)
            compute_stage(s, (1, 0) if s % 2 == 0 else (0, 1))
            partials.append(start_partial(s))
        for r in partials:
            r.wait()

        out_ref[...] = (
            out_ref[...].astype(jnp.float32)
            + precv[0].astype(jnp.float32)
            + precv[1].astype(jnp.float32)
            + precv[2].astype(jnp.float32)
        ).astype(jnp.bfloat16)

    out_bf = pl.pallas_call(
        body,
        out_shape=jax.ShapeDtypeStruct((T, D), jnp.bfloat16),
        in_specs=[
            pl.BlockSpec(memory_space=pltpu.VMEM),
            pl.BlockSpec(memory_space=pltpu.VMEM),
            pl.BlockSpec(memory_space=pltpu.MemorySpace.HBM),
        ],
        out_specs=pl.BlockSpec(memory_space=pltpu.VMEM),
        scratch_shapes=[
            pltpu.VMEM((N_DEV - 1, T, D), jnp.bfloat16),
            pltpu.VMEM((N_DEV - 1, T, 4), jnp.bfloat16),
            pltpu.VMEM((KH, D), jnp.bfloat16),
            pltpu.VMEM((2, TT, KH), jnp.bfloat16),
            pltpu.VMEM((T, D), jnp.bfloat16),
            pltpu.VMEM((N_DEV - 1, T, D), jnp.bfloat16),
            pltpu.VMEM((N_DEV - 1, T, D), jnp.bfloat16),
            pltpu.SemaphoreType.DMA((N_DEV - 1, 2)),
            pltpu.SemaphoreType.DMA((N_DEV - 1, 2)),
            pltpu.SemaphoreType.DMA((N_DEV - 1,)),
            pltpu.SemaphoreType.DMA((N_DEV - 1,)),
            pltpu.SemaphoreType.DMA,
        ],
        compiler_params=pltpu.CompilerParams(
            collective_id=0,
            vmem_limit_bytes=128 * 1024 * 1024,
        ),
    )(x_bf, packed, ew_bf)
    return out_bf.astype(jnp.float32)
